# Initial kernel scaffold; baseline (speedup 1.0000x reference)
import jax
import jax.numpy as jnp
from jax.experimental import pallas as pl


def kernel(x_a_cat, x_a_num, x_t, edge_a2t, edge_t2a, emb_cat, num_w, num_b, aWq, aWk, aWv, aWo, abq, abk, abv, abo, tWq, tWk, tWv, tWo, tbq, tbk, tbv, tbo, s1Wl, s1bl, s1Wr, s2Wl, s2bl, s2Wr, out_w, out_b):
    raise NotImplementedError("write your pallas kernel here")


# speedup vs baseline: 51.1290x; 51.1290x over previous
import functools

import jax
import jax.numpy as jnp
from jax import lax
from jax.experimental import pallas as pl
from jax.experimental.pallas import tpu as pltpu
from jax.experimental.pallas import tpu_sc as plsc

D = 128
NT = 10000
E = 160000
OUT = 16

_NSUB = 16
_EPS = E // _NSUB
_NTP = 10240
_STRIDE = _NTP // _NSUB


def _tc_body(aWv, abv, aWo, abo, tWv, tbv, tWo, tbo,
             s1Wl, s1bl, s1Wr, out_w, out_b, out_ref):
    va = jnp.sum(aWv[...], axis=0, keepdims=True) + abv[...]
    u = jnp.dot(va, aWo[...], preferred_element_type=jnp.float32) + abo[...]
    vt = jnp.sum(tWv[...], axis=0, keepdims=True) + tbv[...]
    ct = jnp.dot(vt, tWo[...], preferred_element_type=jnp.float32) + tbo[...]
    c = ct[0:1, 0:1]
    base = s1bl[...] + c * s1Wr[...]
    row1 = jnp.dot(u, s1Wl[...], preferred_element_type=jnp.float32) + base
    rows = jnp.concatenate([base, row1], axis=0)
    logits = jnp.dot(rows, out_w[...],
                     preferred_element_type=jnp.float32) + out_b[...]
    m = jnp.max(logits, axis=1, keepdims=True)
    e = jnp.exp(logits - m)
    p = e / jnp.sum(e, axis=1, keepdims=True)
    out_ref[0:1, :] = p[0:1, :]
    out_ref[1:2, :] = p[1:2, :] - p[0:1, :]


def _tc_pd(aWv, abv, aWo, abo, tWv, tbv, tWo, tbo, s1Wl, s1bl, s1Wr,
           out_w, out_b):
    r1 = lambda v: jnp.reshape(v, (1, -1))
    return pl.pallas_call(
        _tc_body,
        out_shape=jax.ShapeDtypeStruct((2, OUT), jnp.float32),
    )(aWv, r1(abv), aWo, r1(abo), tWv, r1(tbv), tWo, r1(tbo),
      s1Wl, r1(s1bl), s1Wr, out_w, r1(out_b))


def _sc_body(dst_hbm, pd_hbm, out_hbm, idx_v, flags_v, pd_v, strip_v,
             outblk_v, shared):
    t = lax.axis_index("s")

    pltpu.sync_copy(dst_hbm.at[pl.ds(t * _EPS, _EPS)], idx_v)
    pltpu.sync_copy(pd_hbm, pd_v)
    p0 = pd_v[0, :]
    dv = pd_v[1, :]

    zero16 = jnp.zeros((16,), jnp.float32)
    ones16 = jnp.ones((16,), jnp.float32)

    def _zero(i, carry):
        flags_v[pl.ds(i * 16, 16)] = zero16
        return carry
    lax.fori_loop(0, _NTP // 16, _zero, 0)

    def _scatter(i, carry):
        iv = idx_v[pl.ds(i * 16, 16)]
        plsc.store_scatter(flags_v, [iv], ones16)
        return carry
    lax.fori_loop(0, _EPS // 16, _scatter, 0)

    pltpu.sync_copy(flags_v, shared.at[t])
    plsc.subcore_barrier()

    base = t * _STRIDE
    for r in range(_NSUB):
        pltpu.sync_copy(shared.at[r, pl.ds(base, _STRIDE)], strip_v.at[r])

    def _chunk(c, carry):
        acc = strip_v[0, pl.ds(c * 16, 16)]
        for r in range(1, _NSUB):
            acc = acc + strip_v[r, pl.ds(c * 16, 16)]
        ind = jnp.minimum(acc, 1.0)
        for n in range(16):
            outblk_v[c * 16 + n, :] = p0 + ind[n] * dv
        return carry
    lax.fori_loop(0, _STRIDE // 16, _chunk, 0)

    tail = NT - (_NSUB - 1) * _STRIDE
    @pl.when(t < _NSUB - 1)
    def _():
        pltpu.sync_copy(outblk_v, out_hbm.at[pl.ds(base, _STRIDE), :])
    @pl.when(t == _NSUB - 1)
    def _():
        pltpu.sync_copy(outblk_v.at[pl.ds(0, tail), :],
                        out_hbm.at[pl.ds(base, tail), :])


def _sc_scatter_build():
    mesh = plsc.VectorSubcoreMesh(core_axis_name="c", subcore_axis_name="s")
    return pl.kernel(
        _sc_body,
        mesh=mesh,
        compiler_params=pltpu.CompilerParams(needs_layout_passes=False),
        out_type=jax.ShapeDtypeStruct((NT, OUT), jnp.float32),
        scratch_types=[
            pltpu.VMEM((_EPS,), jnp.int32),
            pltpu.VMEM((_NTP,), jnp.float32),
            pltpu.VMEM((2, OUT), jnp.float32),
            pltpu.VMEM((_NSUB, _STRIDE), jnp.float32),
            pltpu.VMEM((_STRIDE, OUT), jnp.float32),
            pltpu.VMEM_SHARED((_NSUB, _NTP), jnp.float32),
        ],
    )


def kernel(x_a_cat, x_a_num, x_t, edge_a2t, edge_t2a, emb_cat, num_w, num_b,
           aWq, aWk, aWv, aWo, abq, abk, abv, abo,
           tWq, tWk, tWv, tWo, tbq, tbk, tbv, tbo,
           s1Wl, s1bl, s1Wr, s2Wl, s2bl, s2Wr, out_w, out_b):
    pd = _tc_pd(aWv, abv, aWo, abo, tWv, tbv, tWo, tbo,
                s1Wl, s1bl, s1Wr, out_w, out_b)
    dst = edge_a2t[1].astype(jnp.int32)
    return _sc_scatter_build()(dst, pd)


# speedup vs baseline: 61.8726x; 1.2101x over previous
import functools

import jax
import jax.numpy as jnp
from jax import lax
from jax.experimental import pallas as pl
from jax.experimental.pallas import tpu as pltpu
from jax.experimental.pallas import tpu_sc as plsc

D = 128
NT = 10000
E = 160000
OUT = 16

_NSUB = 16
_EPS = E // _NSUB
_NTP = 10240
_STRIDE = _NTP // _NSUB


def _tc_body(aWv, abv, aWo, abo, tWv, tbv, tWo, tbo,
             s1Wl, s1bl, s1Wr, out_w, out_b, out_ref):
    va = jnp.sum(aWv[...], axis=0, keepdims=True) + abv[...]
    u = jnp.dot(va, aWo[...], preferred_element_type=jnp.float32) + abo[...]
    vt = jnp.sum(tWv[...], axis=0, keepdims=True) + tbv[...]
    ct = jnp.dot(vt, tWo[...], preferred_element_type=jnp.float32) + tbo[...]
    c = ct[0:1, 0:1]
    base = s1bl[...] + c * s1Wr[...]
    row1 = jnp.dot(u, s1Wl[...], preferred_element_type=jnp.float32) + base
    rows = jnp.concatenate([base, row1], axis=0)
    logits = jnp.dot(rows, out_w[...],
                     preferred_element_type=jnp.float32) + out_b[...]
    m = jnp.max(logits, axis=1, keepdims=True)
    e = jnp.exp(logits - m)
    p = e / jnp.sum(e, axis=1, keepdims=True)
    out_ref[0:1, :] = p[0:1, :]
    out_ref[1:2, :] = p[1:2, :] - p[0:1, :]


def _tc_pd(aWv, abv, aWo, abo, tWv, tbv, tWo, tbo, s1Wl, s1bl, s1Wr,
           out_w, out_b):
    r1 = lambda v: jnp.reshape(v, (1, -1))
    return pl.pallas_call(
        _tc_body,
        out_shape=jax.ShapeDtypeStruct((2, OUT), jnp.float32),
    )(aWv, r1(abv), aWo, r1(abo), tWv, r1(tbv), tWo, r1(tbo),
      s1Wl, r1(s1bl), s1Wr, out_w, r1(out_b))


def _sc_body(edge_hbm, pd_hbm, out_hbm, idx_v, flags_v, pd_v, strip_v,
             outblk_v, shared, sem):
    t = lax.axis_index("s")

    cp = pltpu.make_async_copy(edge_hbm.at[pl.ds(E + t * _EPS, _EPS)],
                               idx_v, sem)
    cp.start()
    pltpu.sync_copy(pd_hbm, pd_v)
    p0 = pd_v[0, :]
    dv = pd_v[1, :]

    zero16 = jnp.zeros((16,), jnp.float32)
    ones16 = jnp.ones((16,), jnp.float32)

    def _zero(i, carry):
        for j in range(16):
            flags_v[pl.ds((i * 16 + j) * 16, 16)] = zero16
        return carry
    lax.fori_loop(0, _NTP // 256, _zero, 0)
    cp.wait()

    def _scatter(i, carry):
        for j in range(5):
            iv = idx_v[pl.ds((i * 5 + j) * 16, 16)]
            plsc.store_scatter(flags_v, [iv], ones16)
        return carry
    lax.fori_loop(0, _EPS // 80, _scatter, 0)

    pltpu.sync_copy(flags_v, shared.at[t])
    plsc.subcore_barrier()

    base = t * _STRIDE
    for r in range(_NSUB):
        pltpu.sync_copy(shared.at[r, pl.ds(base, _STRIDE)], strip_v.at[r])

    def _chunk(c, carry):
        acc = strip_v[0, pl.ds(c * 16, 16)]
        for r in range(1, _NSUB):
            acc = acc + strip_v[r, pl.ds(c * 16, 16)]
        ind = jnp.minimum(acc, 1.0)
        for n in range(16):
            outblk_v[c * 16 + n, :] = p0 + ind[n] * dv
        return carry
    lax.fori_loop(0, _STRIDE // 16, _chunk, 0)

    tail = NT - (_NSUB - 1) * _STRIDE
    @pl.when(t < _NSUB - 1)
    def _():
        pltpu.sync_copy(outblk_v, out_hbm.at[pl.ds(base, _STRIDE), :])
    @pl.when(t == _NSUB - 1)
    def _():
        pltpu.sync_copy(outblk_v.at[pl.ds(0, tail), :],
                        out_hbm.at[pl.ds(base, tail), :])


def _sc_scatter_build():
    mesh = plsc.VectorSubcoreMesh(core_axis_name="c", subcore_axis_name="s")
    return pl.kernel(
        _sc_body,
        mesh=mesh,
        compiler_params=pltpu.CompilerParams(needs_layout_passes=False),
        out_type=jax.ShapeDtypeStruct((NT, OUT), jnp.float32),
        scratch_types=[
            pltpu.VMEM((_EPS,), jnp.int32),
            pltpu.VMEM((_NTP,), jnp.float32),
            pltpu.VMEM((2, OUT), jnp.float32),
            pltpu.VMEM((_NSUB, _STRIDE), jnp.float32),
            pltpu.VMEM((_STRIDE, OUT), jnp.float32),
            pltpu.VMEM_SHARED((_NSUB, _NTP), jnp.float32),
            pltpu.SemaphoreType.DMA,
        ],
    )


def kernel(x_a_cat, x_a_num, x_t, edge_a2t, edge_t2a, emb_cat, num_w, num_b,
           aWq, aWk, aWv, aWo, abq, abk, abv, abo,
           tWq, tWk, tWv, tWo, tbq, tbk, tbv, tbo,
           s1Wl, s1bl, s1Wr, s2Wl, s2bl, s2Wr, out_w, out_b):
    pd = _tc_pd(aWv, abv, aWo, abo, tWv, tbv, tWo, tbo,
                s1Wl, s1bl, s1Wr, out_w, out_b)
    return _sc_scatter_build()(
        jnp.reshape(edge_a2t.astype(jnp.int32), (2 * E,)), pd)


# speedup vs baseline: 69.3463x; 1.1208x over previous
import functools

import jax
import jax.numpy as jnp
from jax import lax
from jax.experimental import pallas as pl
from jax.experimental.pallas import tpu as pltpu
from jax.experimental.pallas import tpu_sc as plsc

D = 128
NT = 10000
E = 160000
OUT = 16

_NSUB = 16
_EPS = E // _NSUB
_NTP = 10240
_STRIDE = _NTP // _NSUB


def _tc_body(edge, aWv, abv, aWo, abo, tWv, tbv, tWo, tbo,
             s1Wl, s1bl, s1Wr, out_w, out_b, out_ref, dst_ref):
    dst_ref[...] = edge[1, :]
    va = jnp.sum(aWv[...], axis=0, keepdims=True) + abv[...]
    u = jnp.dot(va, aWo[...], preferred_element_type=jnp.float32) + abo[...]
    vt = jnp.sum(tWv[...], axis=0, keepdims=True) + tbv[...]
    ct = jnp.dot(vt, tWo[...], preferred_element_type=jnp.float32) + tbo[...]
    c = ct[0:1, 0:1]
    base = s1bl[...] + c * s1Wr[...]
    row1 = jnp.dot(u, s1Wl[...], preferred_element_type=jnp.float32) + base
    rows = jnp.concatenate([base, row1], axis=0)
    logits = jnp.dot(rows, out_w[...],
                     preferred_element_type=jnp.float32) + out_b[...]
    m = jnp.max(logits, axis=1, keepdims=True)
    e = jnp.exp(logits - m)
    p = e / jnp.sum(e, axis=1, keepdims=True)
    out_ref[0:1, :] = p[0:1, :]
    out_ref[1:2, :] = p[1:2, :] - p[0:1, :]


def _tc_pd(edge, aWv, abv, aWo, abo, tWv, tbv, tWo, tbo, s1Wl, s1bl, s1Wr,
           out_w, out_b):
    r1 = lambda v: jnp.reshape(v, (1, -1))
    return pl.pallas_call(
        _tc_body,
        out_shape=[jax.ShapeDtypeStruct((2, OUT), jnp.float32),
                   jax.ShapeDtypeStruct((E,), jnp.int32)],
    )(edge, aWv, r1(abv), aWo, r1(abo), tWv, r1(tbv), tWo, r1(tbo),
      s1Wl, r1(s1bl), s1Wr, out_w, r1(out_b))


def _sc_body(dst_hbm, pd_hbm, out_hbm, idx_v, flags_v, pd_v, strip_v,
             outblk_v, shared, sem):
    t = lax.axis_index("s")

    cp = pltpu.make_async_copy(dst_hbm.at[pl.ds(t * _EPS, _EPS)], idx_v, sem)
    cp.start()
    pltpu.sync_copy(pd_hbm, pd_v)
    p0 = pd_v[0, :]
    dv = pd_v[1, :]

    zero16 = jnp.zeros((16,), jnp.float32)
    ones16 = jnp.ones((16,), jnp.float32)

    def _zero(i, carry):
        for j in range(16):
            flags_v[pl.ds((i * 16 + j) * 16, 16)] = zero16
        return carry
    lax.fori_loop(0, _NTP // 256, _zero, 0)
    cp.wait()

    def _scatter(i, carry):
        for j in range(5):
            iv = idx_v[pl.ds((i * 5 + j) * 16, 16)]
            plsc.store_scatter(flags_v, [iv], ones16)
        return carry
    lax.fori_loop(0, _EPS // 80, _scatter, 0)

    pltpu.sync_copy(flags_v, shared.at[t])
    plsc.subcore_barrier()

    base = t * _STRIDE
    pltpu.sync_copy(shared.at[:, pl.ds(base, _STRIDE)], strip_v)

    def _chunk(c, carry):
        acc = strip_v[0, pl.ds(c * 16, 16)]
        for r in range(1, _NSUB):
            acc = acc + strip_v[r, pl.ds(c * 16, 16)]
        ind = jnp.minimum(acc, 1.0)
        for n in range(16):
            outblk_v[c * 16 + n, :] = p0 + ind[n] * dv
        return carry
    lax.fori_loop(0, _STRIDE // 16, _chunk, 0)

    tail = NT - (_NSUB - 1) * _STRIDE
    @pl.when(t < _NSUB - 1)
    def _():
        pltpu.sync_copy(outblk_v, out_hbm.at[pl.ds(base, _STRIDE), :])
    @pl.when(t == _NSUB - 1)
    def _():
        pltpu.sync_copy(outblk_v.at[pl.ds(0, tail), :],
                        out_hbm.at[pl.ds(base, tail), :])


def _sc_scatter_build():
    mesh = plsc.VectorSubcoreMesh(core_axis_name="c", subcore_axis_name="s")
    return pl.kernel(
        _sc_body,
        mesh=mesh,
        compiler_params=pltpu.CompilerParams(needs_layout_passes=False),
        out_type=jax.ShapeDtypeStruct((NT, OUT), jnp.float32),
        scratch_types=[
            pltpu.VMEM((_EPS,), jnp.int32),
            pltpu.VMEM((_NTP,), jnp.float32),
            pltpu.VMEM((2, OUT), jnp.float32),
            pltpu.VMEM((_NSUB, _STRIDE), jnp.float32),
            pltpu.VMEM((_STRIDE, OUT), jnp.float32),
            pltpu.VMEM_SHARED((_NSUB, _NTP), jnp.float32),
            pltpu.SemaphoreType.DMA,
        ],
    )


def kernel(x_a_cat, x_a_num, x_t, edge_a2t, edge_t2a, emb_cat, num_w, num_b,
           aWq, aWk, aWv, aWo, abq, abk, abv, abo,
           tWq, tWk, tWv, tWo, tbq, tbk, tbv, tbo,
           s1Wl, s1bl, s1Wr, s2Wl, s2bl, s2Wr, out_w, out_b):
    pd, dst = _tc_pd(edge_a2t.astype(jnp.int32), aWv, abv, aWo, abo,
                     tWv, tbv, tWo, tbo, s1Wl, s1bl, s1Wr, out_w, out_b)
    return _sc_scatter_build()(dst, pd)


# speedup vs baseline: 77.4213x; 1.1164x over previous
import functools

import jax
import jax.numpy as jnp
from jax import lax
from jax.experimental import pallas as pl
from jax.experimental.pallas import tpu as pltpu
from jax.experimental.pallas import tpu_sc as plsc

D = 128
NT = 10000
E = 160000
OUT = 16

_NSUB = 16
_EPS = E // _NSUB
_NTP = 10240
_STRIDE = _NTP // _NSUB


def _tc_body(edge, aWv, abv, aWo, abo, tWv, tbv, tWo, tbo,
             s1Wl, s1bl, s1Wr, out_w, out_b, out_ref, dst_ref):
    dst_ref[...] = edge[1, :]
    va = jnp.sum(aWv[...], axis=0, keepdims=True) + abv[...]
    u = jnp.dot(va, aWo[...], preferred_element_type=jnp.float32) + abo[...]
    vt = jnp.sum(tWv[...], axis=0, keepdims=True) + tbv[...]
    ct = jnp.dot(vt, tWo[...], preferred_element_type=jnp.float32) + tbo[...]
    c = ct[0:1, 0:1]
    base = s1bl[...] + c * s1Wr[...]
    row1 = jnp.dot(u, s1Wl[...], preferred_element_type=jnp.float32) + base
    rows = jnp.concatenate([base, row1], axis=0)
    logits = jnp.dot(rows, out_w[...],
                     preferred_element_type=jnp.float32) + out_b[...]
    m = jnp.max(logits, axis=1, keepdims=True)
    e = jnp.exp(logits - m)
    p = e / jnp.sum(e, axis=1, keepdims=True)
    out_ref[0:1, :] = p[0:1, :]
    out_ref[1:2, :] = p[1:2, :] - p[0:1, :]


def _tc_pd(edge, aWv, abv, aWo, abo, tWv, tbv, tWo, tbo, s1Wl, s1bl, s1Wr,
           out_w, out_b):
    r1 = lambda v: jnp.reshape(v, (1, -1))
    return pl.pallas_call(
        _tc_body,
        out_shape=[jax.ShapeDtypeStruct((2, OUT), jnp.float32),
                   jax.ShapeDtypeStruct((E,), jnp.int32)],
    )(edge, aWv, r1(abv), aWo, r1(abo), tWv, r1(tbv), tWo, r1(tbo),
      s1Wl, r1(s1bl), s1Wr, out_w, r1(out_b))


def _sc_body(dst_hbm, pd_hbm, out_hbm, idx_v, flags_v, pd_v, strip_v,
             outblk_v, shared, sem):
    t = lax.axis_index("s")

    cp = pltpu.make_async_copy(dst_hbm.at[pl.ds(t * _EPS, _EPS)], idx_v, sem)
    cp.start()
    pltpu.sync_copy(pd_hbm, pd_v)
    p0 = pd_v[0, :]
    dv = pd_v[1, :]

    zero16 = jnp.zeros((16,), jnp.float32)
    ones16 = jnp.ones((16,), jnp.float32)

    @plsc.parallel_loop(0, _NTP // 16, unroll=8)
    def _zero(i):
        flags_v[pl.ds(i * 16, 16)] = zero16
    cp.wait()

    @plsc.parallel_loop(0, _EPS // 16, unroll=8)
    def _scatter(i):
        iv = idx_v[pl.ds(i * 16, 16)]
        plsc.store_scatter(flags_v, [iv], ones16)

    pltpu.sync_copy(flags_v, shared.at[t])
    plsc.subcore_barrier()

    base = t * _STRIDE
    pltpu.sync_copy(shared.at[:, pl.ds(base, _STRIDE)], strip_v)

    @plsc.parallel_loop(0, _STRIDE // 16, unroll=2)
    def _chunk(c):
        acc = strip_v[0, pl.ds(c * 16, 16)]
        for r in range(1, _NSUB):
            acc = acc + strip_v[r, pl.ds(c * 16, 16)]
        ind = jnp.minimum(acc, 1.0)
        for n in range(16):
            outblk_v[c * 16 + n, :] = p0 + ind[n] * dv

    tail = NT - (_NSUB - 1) * _STRIDE
    @pl.when(t < _NSUB - 1)
    def _():
        pltpu.sync_copy(outblk_v, out_hbm.at[pl.ds(base, _STRIDE), :])
    @pl.when(t == _NSUB - 1)
    def _():
        pltpu.sync_copy(outblk_v.at[pl.ds(0, tail), :],
                        out_hbm.at[pl.ds(base, tail), :])


def _sc_scatter_build():
    mesh = plsc.VectorSubcoreMesh(core_axis_name="c", subcore_axis_name="s")
    return pl.kernel(
        _sc_body,
        mesh=mesh,
        compiler_params=pltpu.CompilerParams(needs_layout_passes=False),
        out_type=jax.ShapeDtypeStruct((NT, OUT), jnp.float32),
        scratch_types=[
            pltpu.VMEM((_EPS,), jnp.int32),
            pltpu.VMEM((_NTP,), jnp.float32),
            pltpu.VMEM((2, OUT), jnp.float32),
            pltpu.VMEM((_NSUB, _STRIDE), jnp.float32),
            pltpu.VMEM((_STRIDE, OUT), jnp.float32),
            pltpu.VMEM_SHARED((_NSUB, _NTP), jnp.float32),
            pltpu.SemaphoreType.DMA,
        ],
    )


def kernel(x_a_cat, x_a_num, x_t, edge_a2t, edge_t2a, emb_cat, num_w, num_b,
           aWq, aWk, aWv, aWo, abq, abk, abv, abo,
           tWq, tWk, tWv, tWo, tbq, tbk, tbv, tbo,
           s1Wl, s1bl, s1Wr, s2Wl, s2bl, s2Wr, out_w, out_b):
    pd, dst = _tc_pd(edge_a2t.astype(jnp.int32), aWv, abv, aWo, abo,
                     tWv, tbv, tWo, tbo, s1Wl, s1bl, s1Wr, out_w, out_b)
    return _sc_scatter_build()(dst, pd)


# speedup vs baseline: 77.4955x; 1.0010x over previous
import functools

import jax
import jax.numpy as jnp
from jax import lax
from jax.experimental import pallas as pl
from jax.experimental.pallas import tpu as pltpu
from jax.experimental.pallas import tpu_sc as plsc

D = 128
NT = 10000
E = 160000
OUT = 16

_NSUB = 16
_EPS = E // _NSUB
_NTP = 10240
_STRIDE = _NTP // _NSUB


def _tc_body(edge, aWv, abv, aWo, abo, tWv, tbv, tWo, tbo,
             s1Wl, s1bl, s1Wr, out_w, out_b, out_ref, dst_ref):
    dst_ref[...] = edge[1, :]
    va = jnp.sum(aWv[...], axis=0, keepdims=True) + abv[...]
    u = jnp.dot(va, aWo[...], preferred_element_type=jnp.float32) + abo[...]
    vt = jnp.sum(tWv[...], axis=0, keepdims=True) + tbv[...]
    ct = jnp.dot(vt, tWo[...], preferred_element_type=jnp.float32) + tbo[...]
    c = ct[0:1, 0:1]
    base = s1bl[...] + c * s1Wr[...]
    row1 = jnp.dot(u, s1Wl[...], preferred_element_type=jnp.float32) + base
    rows = jnp.concatenate([base, row1], axis=0)
    logits = jnp.dot(rows, out_w[...],
                     preferred_element_type=jnp.float32) + out_b[...]
    m = jnp.max(logits, axis=1, keepdims=True)
    e = jnp.exp(logits - m)
    p = e / jnp.sum(e, axis=1, keepdims=True)
    out_ref[0:1, :] = p[0:1, :]
    out_ref[1:2, :] = p[1:2, :] - p[0:1, :]


def _tc_pd(edge, aWv, abv, aWo, abo, tWv, tbv, tWo, tbo, s1Wl, s1bl, s1Wr,
           out_w, out_b):
    r1 = lambda v: jnp.reshape(v, (1, -1))
    return pl.pallas_call(
        _tc_body,
        out_shape=[jax.ShapeDtypeStruct((2, OUT), jnp.float32),
                   jax.ShapeDtypeStruct((E,), jnp.int32)],
    )(edge, aWv, r1(abv), aWo, r1(abo), tWv, r1(tbv), tWo, r1(tbo),
      s1Wl, r1(s1bl), s1Wr, out_w, r1(out_b))


def _sc_body(dst_hbm, pd_hbm, out_hbm, idx_v, flags_v, pd_v, strip_v,
             outblk_v, shared, sem):
    t = lax.axis_index("s")

    cp = pltpu.make_async_copy(dst_hbm.at[pl.ds(t * _EPS, _EPS)], idx_v, sem)
    cp.start()
    pltpu.sync_copy(pd_hbm, pd_v)
    p0 = pd_v[0, :]
    dv = pd_v[1, :]

    zero16 = jnp.zeros((16,), jnp.float32)
    ones16 = jnp.ones((16,), jnp.float32)

    @plsc.parallel_loop(0, _NTP // 16, unroll=8)
    def _zero(i):
        flags_v[pl.ds(i * 16, 16)] = zero16
    cp.wait()

    @plsc.parallel_loop(0, _EPS // 16, unroll=8)
    def _scatter(i):
        iv = idx_v[pl.ds(i * 16, 16)]
        plsc.store_scatter(flags_v, [iv], ones16)

    pltpu.sync_copy(flags_v, shared.at[t])
    plsc.subcore_barrier()

    base = t * _STRIDE
    pltpu.sync_copy(shared.at[:, pl.ds(base, _STRIDE)], strip_v)

    @plsc.parallel_loop(0, _STRIDE // 16, unroll=2)
    def _chunk(c):
        rows = [strip_v[r, pl.ds(c * 16, 16)] for r in range(_NSUB)]
        while len(rows) > 1:
            rows = [a + b for a, b in zip(rows[::2], rows[1::2])]
        ind = jnp.minimum(rows[0], 1.0)
        for n in range(16):
            outblk_v[c * 16 + n, :] = p0 + ind[n] * dv

    tail = NT - (_NSUB - 1) * _STRIDE
    @pl.when(t < _NSUB - 1)
    def _():
        pltpu.sync_copy(outblk_v, out_hbm.at[pl.ds(base, _STRIDE), :])
    @pl.when(t == _NSUB - 1)
    def _():
        pltpu.sync_copy(outblk_v.at[pl.ds(0, tail), :],
                        out_hbm.at[pl.ds(base, tail), :])


def _sc_scatter_build():
    mesh = plsc.VectorSubcoreMesh(core_axis_name="c", subcore_axis_name="s")
    return pl.kernel(
        _sc_body,
        mesh=mesh,
        compiler_params=pltpu.CompilerParams(needs_layout_passes=False),
        out_type=jax.ShapeDtypeStruct((NT, OUT), jnp.float32),
        scratch_types=[
            pltpu.VMEM((_EPS,), jnp.int32),
            pltpu.VMEM((_NTP,), jnp.float32),
            pltpu.VMEM((2, OUT), jnp.float32),
            pltpu.VMEM((_NSUB, _STRIDE), jnp.float32),
            pltpu.VMEM((_STRIDE, OUT), jnp.float32),
            pltpu.VMEM_SHARED((_NSUB, _NTP), jnp.float32),
            pltpu.SemaphoreType.DMA,
        ],
    )


def kernel(x_a_cat, x_a_num, x_t, edge_a2t, edge_t2a, emb_cat, num_w, num_b,
           aWq, aWk, aWv, aWo, abq, abk, abv, abo,
           tWq, tWk, tWv, tWo, tbq, tbk, tbv, tbo,
           s1Wl, s1bl, s1Wr, s2Wl, s2bl, s2Wr, out_w, out_b):
    pd, dst = _tc_pd(edge_a2t.astype(jnp.int32), aWv, abv, aWo, abo,
                     tWv, tbv, tWo, tbo, s1Wl, s1bl, s1Wr, out_w, out_b)
    return _sc_scatter_build()(dst, pd)


# speedup vs baseline: 77.6117x; 1.0015x over previous
import functools

import jax
import jax.numpy as jnp
from jax import lax
from jax.experimental import pallas as pl
from jax.experimental.pallas import tpu as pltpu
from jax.experimental.pallas import tpu_sc as plsc

D = 128
NT = 10000
E = 160000
OUT = 16

_NSUB = 16
_EPS = E // _NSUB
_NTP = 10240
_STRIDE = _NTP // _NSUB


def _tc_body(edge, aWv, abv, aWo, abo, tWv, tbv, tWo, tbo,
             s1Wl, s1bl, s1Wr, out_w, out_b, out_ref, dst_ref):
    dst_ref[...] = edge[1, :]
    va = jnp.sum(aWv[...], axis=0, keepdims=True) + abv[...]
    u = jnp.dot(va, aWo[...], preferred_element_type=jnp.float32) + abo[...]
    vt = jnp.sum(tWv[...], axis=0, keepdims=True) + tbv[...]
    ct = jnp.dot(vt, tWo[...], preferred_element_type=jnp.float32) + tbo[...]
    c = ct[0:1, 0:1]
    base = s1bl[...] + c * s1Wr[...]
    row1 = jnp.dot(u, s1Wl[...], preferred_element_type=jnp.float32) + base
    rows = jnp.concatenate([base, row1], axis=0)
    logits = jnp.dot(rows, out_w[...],
                     preferred_element_type=jnp.float32) + out_b[...]
    m = jnp.max(logits, axis=1, keepdims=True)
    e = jnp.exp(logits - m)
    p = e / jnp.sum(e, axis=1, keepdims=True)
    out_ref[0:1, :] = p[0:1, :]
    out_ref[1:2, :] = p[1:2, :] - p[0:1, :]


def _tc_pd(edge, aWv, abv, aWo, abo, tWv, tbv, tWo, tbo, s1Wl, s1bl, s1Wr,
           out_w, out_b):
    r1 = lambda v: jnp.reshape(v, (1, -1))
    return pl.pallas_call(
        _tc_body,
        out_shape=[jax.ShapeDtypeStruct((2, OUT), jnp.float32),
                   jax.ShapeDtypeStruct((E,), jnp.int32)],
    )(edge, aWv, r1(abv), aWo, r1(abo), tWv, r1(tbv), tWo, r1(tbo),
      s1Wl, r1(s1bl), s1Wr, out_w, r1(out_b))


def _sc_body(dst_hbm, pd_hbm, out_hbm, idx_v, flags_v, pd_v, strip_v,
             outblk_v, shared, sem):
    t = lax.axis_index("s")

    cp = pltpu.make_async_copy(dst_hbm.at[pl.ds(t * _EPS, _EPS)], idx_v, sem)
    cp.start()
    pltpu.sync_copy(pd_hbm, pd_v)
    p0 = pd_v[0, :]
    dv = pd_v[1, :]

    zero16 = jnp.zeros((16,), jnp.float32)
    ones16 = jnp.ones((16,), jnp.float32)

    with jax.named_scope("ph_zero"):
        @plsc.parallel_loop(0, _NTP // 16, unroll=8)
        def _zero(i):
            flags_v[pl.ds(i * 16, 16)] = zero16
    with jax.named_scope("ph_idxwait"):
        cp.wait()

    with jax.named_scope("ph_scatter"):
        @plsc.parallel_loop(0, _EPS // 16, unroll=8)
        def _scatter(i):
            iv = idx_v[pl.ds(i * 16, 16)]
            plsc.store_scatter(flags_v, [iv], ones16)

    with jax.named_scope("ph_publish"):
        pltpu.sync_copy(flags_v, shared.at[t])
        plsc.subcore_barrier()

    base = t * _STRIDE
    with jax.named_scope("ph_gather"):
        pltpu.sync_copy(shared.at[:, pl.ds(base, _STRIDE)], strip_v)

    with jax.named_scope("ph_assemble"):
        _assemble(strip_v, outblk_v, p0, dv)

    tail = NT - (_NSUB - 1) * _STRIDE
    with jax.named_scope("ph_outwrite"):
        @pl.when(t < _NSUB - 1)
        def _():
            pltpu.sync_copy(outblk_v, out_hbm.at[pl.ds(base, _STRIDE), :])
        @pl.when(t == _NSUB - 1)
        def _():
            pltpu.sync_copy(outblk_v.at[pl.ds(0, tail), :],
                            out_hbm.at[pl.ds(base, tail), :])


def _assemble(strip_v, outblk_v, p0, dv):
    @plsc.parallel_loop(0, _STRIDE // 16, unroll=2)
    def _chunk(c):
        rows = [strip_v[r, pl.ds(c * 16, 16)] for r in range(_NSUB)]
        while len(rows) > 1:
            rows = [a + b for a, b in zip(rows[::2], rows[1::2])]
        ind = jnp.minimum(rows[0], 1.0)
        for n in range(16):
            outblk_v[c * 16 + n, :] = p0 + ind[n] * dv


def _sc_scatter_build():
    mesh = plsc.VectorSubcoreMesh(core_axis_name="c", subcore_axis_name="s")
    return pl.kernel(
        _sc_body,
        mesh=mesh,
        compiler_params=pltpu.CompilerParams(needs_layout_passes=False),
        out_type=jax.ShapeDtypeStruct((NT, OUT), jnp.float32),
        scratch_types=[
            pltpu.VMEM((_EPS,), jnp.int32),
            pltpu.VMEM((_NTP,), jnp.float32),
            pltpu.VMEM((2, OUT), jnp.float32),
            pltpu.VMEM((_NSUB, _STRIDE), jnp.float32),
            pltpu.VMEM((_STRIDE, OUT), jnp.float32),
            pltpu.VMEM_SHARED((_NSUB, _NTP), jnp.float32),
            pltpu.SemaphoreType.DMA,
        ],
    )


def kernel(x_a_cat, x_a_num, x_t, edge_a2t, edge_t2a, emb_cat, num_w, num_b,
           aWq, aWk, aWv, aWo, abq, abk, abv, abo,
           tWq, tWk, tWv, tWo, tbq, tbk, tbv, tbo,
           s1Wl, s1bl, s1Wr, s2Wl, s2bl, s2Wr, out_w, out_b):
    pd, dst = _tc_pd(edge_a2t.astype(jnp.int32), aWv, abv, aWo, abo,
                     tWv, tbv, tWo, tbo, s1Wl, s1bl, s1Wr, out_w, out_b)
    return _sc_scatter_build()(dst, pd)


# speedup vs baseline: 77.9859x; 1.0048x over previous
import functools

import jax
import jax.numpy as jnp
from jax import lax
from jax.experimental import pallas as pl
from jax.experimental.pallas import tpu as pltpu
from jax.experimental.pallas import tpu_sc as plsc

D = 128
NT = 10000
E = 160000
OUT = 16

_NSUB = 16
_EPS = E // _NSUB
_NTP = 10240
_STRIDE = _NTP // _NSUB


def _tc_body(edge, aWv, abv, aWo, abo, tWv, tbv, tWo, tbo,
             s1Wl, s1bl, s1Wr, out_w, out_b, out_ref, dst_ref):
    dst_ref[...] = edge[1, :]
    va = jnp.sum(aWv[...], axis=0, keepdims=True) + abv[...]
    u = jnp.dot(va, aWo[...], preferred_element_type=jnp.float32) + abo[...]
    vt = jnp.sum(tWv[...], axis=0, keepdims=True) + tbv[...]
    ct = jnp.dot(vt, tWo[...], preferred_element_type=jnp.float32) + tbo[...]
    c = ct[0:1, 0:1]
    base = s1bl[...] + c * s1Wr[...]
    row1 = jnp.dot(u, s1Wl[...], preferred_element_type=jnp.float32) + base
    rows = jnp.concatenate([base, row1], axis=0)
    logits = jnp.dot(rows, out_w[...],
                     preferred_element_type=jnp.float32) + out_b[...]
    m = jnp.max(logits, axis=1, keepdims=True)
    e = jnp.exp(logits - m)
    p = e / jnp.sum(e, axis=1, keepdims=True)
    out_ref[0:1, :] = p[0:1, :]
    out_ref[1:2, :] = p[1:2, :] - p[0:1, :]


def _tc_pd(edge, aWv, abv, aWo, abo, tWv, tbv, tWo, tbo, s1Wl, s1bl, s1Wr,
           out_w, out_b):
    r1 = lambda v: jnp.reshape(v, (1, -1))
    return pl.pallas_call(
        _tc_body,
        out_shape=[jax.ShapeDtypeStruct((2, OUT), jnp.float32),
                   jax.ShapeDtypeStruct((E,), jnp.int32)],
    )(edge, aWv, r1(abv), aWo, r1(abo), tWv, r1(tbv), tWo, r1(tbo),
      s1Wl, r1(s1bl), s1Wr, out_w, r1(out_b))


def _sc_body(dst_hbm, pd_hbm, out_hbm, idx_v, flags_v, pd_v, strip_v,
             outblk_v, shared, sem):
    t = lax.axis_index("s")

    cp = pltpu.make_async_copy(dst_hbm.at[pl.ds(t * _EPS, _EPS)], idx_v, sem)
    cp.start()
    pltpu.sync_copy(pd_hbm, pd_v)
    p0 = pd_v[0, :]
    dv = pd_v[1, :]

    zero16 = jnp.zeros((16,), jnp.float32)
    ones16 = jnp.ones((16,), jnp.float32)

    with jax.named_scope("ph_zero"):
        @plsc.parallel_loop(0, _NTP // 16, unroll=8)
        def _zero(i):
            flags_v[pl.ds(i * 16, 16)] = zero16
    with jax.named_scope("ph_idxwait"):
        cp.wait()

    with jax.named_scope("ph_scatter"):
        @plsc.parallel_loop(0, _EPS // 16, unroll=8)
        def _scatter(i):
            iv = idx_v[pl.ds(i * 16, 16)]
            plsc.store_scatter(flags_v, [iv], ones16)

    with jax.named_scope("ph_publish"):
        pltpu.sync_copy(flags_v, shared.at[t])
        plsc.subcore_barrier()

    base = t * _STRIDE
    with jax.named_scope("ph_gather"):
        pltpu.sync_copy(shared.at[:, pl.ds(base, _STRIDE)], strip_v)

    with jax.named_scope("ph_assemble"):
        _assemble(strip_v, outblk_v, p0, dv)

    cid = lax.axis_index("c")
    tail = NT - (_NSUB - 1) * _STRIDE
    mine = (t < _NSUB // 2) == (cid == 0)
    with jax.named_scope("ph_outwrite"):
        @pl.when(mine & (t < _NSUB - 1))
        def _():
            pltpu.sync_copy(outblk_v, out_hbm.at[pl.ds(base, _STRIDE), :])
        @pl.when(mine & (t == _NSUB - 1))
        def _():
            pltpu.sync_copy(outblk_v.at[pl.ds(0, tail), :],
                            out_hbm.at[pl.ds(base, tail), :])


def _assemble(strip_v, outblk_v, p0, dv):
    @plsc.parallel_loop(0, _STRIDE // 16, unroll=2)
    def _chunk(c):
        rows = [strip_v[r, pl.ds(c * 16, 16)] for r in range(_NSUB)]
        while len(rows) > 1:
            rows = [a + b for a, b in zip(rows[::2], rows[1::2])]
        ind = jnp.minimum(rows[0], 1.0)
        for n in range(16):
            outblk_v[c * 16 + n, :] = p0 + ind[n] * dv


def _sc_scatter_build():
    mesh = plsc.VectorSubcoreMesh(core_axis_name="c", subcore_axis_name="s")
    return pl.kernel(
        _sc_body,
        mesh=mesh,
        compiler_params=pltpu.CompilerParams(needs_layout_passes=False),
        out_type=jax.ShapeDtypeStruct((NT, OUT), jnp.float32),
        scratch_types=[
            pltpu.VMEM((_EPS,), jnp.int32),
            pltpu.VMEM((_NTP,), jnp.float32),
            pltpu.VMEM((2, OUT), jnp.float32),
            pltpu.VMEM((_NSUB, _STRIDE), jnp.float32),
            pltpu.VMEM((_STRIDE, OUT), jnp.float32),
            pltpu.VMEM_SHARED((_NSUB, _NTP), jnp.float32),
            pltpu.SemaphoreType.DMA,
        ],
    )


def kernel(x_a_cat, x_a_num, x_t, edge_a2t, edge_t2a, emb_cat, num_w, num_b,
           aWq, aWk, aWv, aWo, abq, abk, abv, abo,
           tWq, tWk, tWv, tWo, tbq, tbk, tbv, tbo,
           s1Wl, s1bl, s1Wr, s2Wl, s2bl, s2Wr, out_w, out_b):
    pd, dst = _tc_pd(edge_a2t.astype(jnp.int32), aWv, abv, aWo, abo,
                     tWv, tbv, tWo, tbo, s1Wl, s1bl, s1Wr, out_w, out_b)
    return _sc_scatter_build()(dst, pd)


# speedup vs baseline: 78.0450x; 1.0008x over previous
import functools

import jax
import jax.numpy as jnp
from jax import lax
from jax.experimental import pallas as pl
from jax.experimental.pallas import tpu as pltpu
from jax.experimental.pallas import tpu_sc as plsc

D = 128
NT = 10000
E = 160000
OUT = 16

_NSUB = 16
_EPS = E // _NSUB
_NTP = 10240
_STRIDE = _NTP // _NSUB


def _tc_body(edge, aWv, abv, aWo, abo, tWv, tbv, tWo, tbo,
             s1Wl, s1bl, s1Wr, out_w, out_b, out_ref, dst_ref):
    dst_ref[...] = edge[1, :]
    va = jnp.sum(aWv[...], axis=0, keepdims=True) + abv[...]
    u = jnp.dot(va, aWo[...], preferred_element_type=jnp.float32) + abo[...]
    vt = jnp.sum(tWv[...], axis=0, keepdims=True) + tbv[...]
    ct = jnp.dot(vt, tWo[...], preferred_element_type=jnp.float32) + tbo[...]
    c = ct[0:1, 0:1]
    base = s1bl[...] + c * s1Wr[...]
    row1 = jnp.dot(u, s1Wl[...], preferred_element_type=jnp.float32) + base
    rows = jnp.concatenate([base, row1], axis=0)
    logits = jnp.dot(rows, out_w[...],
                     preferred_element_type=jnp.float32) + out_b[...]
    m = jnp.max(logits, axis=1, keepdims=True)
    e = jnp.exp(logits - m)
    p = e / jnp.sum(e, axis=1, keepdims=True)
    out_ref[0:1, :] = p[0:1, :]
    out_ref[1:2, :] = p[1:2, :] - p[0:1, :]


def _tc_pd(edge, aWv, abv, aWo, abo, tWv, tbv, tWo, tbo, s1Wl, s1bl, s1Wr,
           out_w, out_b):
    r1 = lambda v: jnp.reshape(v, (1, -1))
    return pl.pallas_call(
        _tc_body,
        out_shape=[jax.ShapeDtypeStruct((2, OUT), jnp.float32),
                   jax.ShapeDtypeStruct((E,), jnp.int32)],
    )(edge, aWv, r1(abv), aWo, r1(abo), tWv, r1(tbv), tWo, r1(tbo),
      s1Wl, r1(s1bl), s1Wr, out_w, r1(out_b))


def _sc_body(dst_hbm, pd_hbm, out_hbm, idx_v, flags_v, pd_v, strip_v,
             outblk_v, shared, sem):
    t = lax.axis_index("s")

    cp = pltpu.make_async_copy(dst_hbm.at[pl.ds(t * _EPS, _EPS)], idx_v, sem)
    cp.start()
    pltpu.sync_copy(pd_hbm, pd_v)
    p0 = pd_v[0, :]
    dv = pd_v[1, :]

    zero16 = jnp.zeros((16,), jnp.float32)
    ones16 = jnp.ones((16,), jnp.float32)

    with jax.named_scope("ph_zero"):
        @plsc.parallel_loop(0, _NTP // 16, unroll=8)
        def _zero(i):
            flags_v[pl.ds(i * 16, 16)] = zero16
    with jax.named_scope("ph_idxwait"):
        cp.wait()

    with jax.named_scope("ph_scatter"):
        @plsc.parallel_loop(0, _EPS // 16, unroll=8)
        def _scatter(i):
            iv = idx_v[pl.ds(i * 16, 16)]
            plsc.store_scatter(flags_v, [iv], ones16)

    with jax.named_scope("ph_publish"):
        pltpu.sync_copy(flags_v, shared.at[t])
        plsc.subcore_barrier()

    base = t * _STRIDE
    with jax.named_scope("ph_gather"):
        pltpu.sync_copy(shared.at[:, pl.ds(base, _STRIDE)], strip_v)

    with jax.named_scope("ph_assemble"):
        _assemble(strip_v, outblk_v, p0, dv)

    cid = lax.axis_index("c")
    tail = NT - (_NSUB - 1) * _STRIDE
    mine = (t < _NSUB // 2) == (cid == 0)
    with jax.named_scope("ph_outwrite"):
        @pl.when(mine & (t < _NSUB - 1))
        def _():
            q = _STRIDE // 4
            cps = [pltpu.make_async_copy(
                outblk_v.at[pl.ds(i * q, q), :],
                out_hbm.at[pl.ds(base + i * q, q), :], sem) for i in range(4)]
            for c in cps:
                c.start()
            for c in cps:
                c.wait()
        @pl.when(mine & (t == _NSUB - 1))
        def _():
            q = tail // 2
            cps = [pltpu.make_async_copy(
                outblk_v.at[pl.ds(i * q, q), :],
                out_hbm.at[pl.ds(base + i * q, q), :], sem) for i in range(2)]
            for c in cps:
                c.start()
            for c in cps:
                c.wait()


def _assemble(strip_v, outblk_v, p0, dv):
    @plsc.parallel_loop(0, _STRIDE // 16, unroll=2)
    def _chunk(c):
        rows = [strip_v[r, pl.ds(c * 16, 16)] for r in range(_NSUB)]
        while len(rows) > 1:
            rows = [a + b for a, b in zip(rows[::2], rows[1::2])]
        ind = jnp.minimum(rows[0], 1.0)
        for n in range(16):
            outblk_v[c * 16 + n, :] = p0 + ind[n] * dv


def _sc_scatter_build():
    mesh = plsc.VectorSubcoreMesh(core_axis_name="c", subcore_axis_name="s")
    return pl.kernel(
        _sc_body,
        mesh=mesh,
        compiler_params=pltpu.CompilerParams(needs_layout_passes=False),
        out_type=jax.ShapeDtypeStruct((NT, OUT), jnp.float32),
        scratch_types=[
            pltpu.VMEM((_EPS,), jnp.int32),
            pltpu.VMEM((_NTP,), jnp.float32),
            pltpu.VMEM((2, OUT), jnp.float32),
            pltpu.VMEM((_NSUB, _STRIDE), jnp.float32),
            pltpu.VMEM((_STRIDE, OUT), jnp.float32),
            pltpu.VMEM_SHARED((_NSUB, _NTP), jnp.float32),
            pltpu.SemaphoreType.DMA,
        ],
    )


def kernel(x_a_cat, x_a_num, x_t, edge_a2t, edge_t2a, emb_cat, num_w, num_b,
           aWq, aWk, aWv, aWo, abq, abk, abv, abo,
           tWq, tWk, tWv, tWo, tbq, tbk, tbv, tbo,
           s1Wl, s1bl, s1Wr, s2Wl, s2bl, s2Wr, out_w, out_b):
    pd, dst = _tc_pd(edge_a2t.astype(jnp.int32), aWv, abv, aWo, abo,
                     tWv, tbv, tWo, tbo, s1Wl, s1bl, s1Wr, out_w, out_b)
    return _sc_scatter_build()(dst, pd)


# speedup vs baseline: 81.6439x; 1.0461x over previous
import functools

import jax
import jax.numpy as jnp
from jax import lax
from jax.experimental import pallas as pl
from jax.experimental.pallas import tpu as pltpu
from jax.experimental.pallas import tpu_sc as plsc

D = 128
NT = 10000
E = 160000
OUT = 16

_NSUB = 16
_EPS = E // _NSUB
_NTP = 10240
_STRIDE = _NTP // _NSUB


def _tc_body(edge, aWv, abv, aWo, abo, tWv, tbv, tWo, tbo,
             s1Wl, s1bl, s1Wr, out_w, out_b, out_ref, dst_ref):
    dst_ref[...] = edge[1, :]
    va = jnp.sum(aWv[...], axis=0, keepdims=True) + abv[...]
    u = jnp.dot(va, aWo[...], preferred_element_type=jnp.float32) + abo[...]
    vt = jnp.sum(tWv[...], axis=0, keepdims=True) + tbv[...]
    ct = jnp.dot(vt, tWo[...], preferred_element_type=jnp.float32) + tbo[...]
    c = ct[0:1, 0:1]
    base = s1bl[...] + c * s1Wr[...]
    row1 = jnp.dot(u, s1Wl[...], preferred_element_type=jnp.float32) + base
    rows = jnp.concatenate([base, row1], axis=0)
    logits = jnp.dot(rows, out_w[...],
                     preferred_element_type=jnp.float32) + out_b[...]
    m = jnp.max(logits, axis=1, keepdims=True)
    e = jnp.exp(logits - m)
    p = e / jnp.sum(e, axis=1, keepdims=True)
    out_ref[0:1, :] = p[0:1, :]
    out_ref[1:2, :] = p[1:2, :] - p[0:1, :]


def _tc_pd(edge, aWv, abv, aWo, abo, tWv, tbv, tWo, tbo, s1Wl, s1bl, s1Wr,
           out_w, out_b):
    r1 = lambda v: jnp.reshape(v, (1, -1))
    return pl.pallas_call(
        _tc_body,
        out_shape=[jax.ShapeDtypeStruct((2, OUT), jnp.float32),
                   jax.ShapeDtypeStruct((E,), jnp.int32)],
    )(edge, aWv, r1(abv), aWo, r1(abo), tWv, r1(tbv), tWo, r1(tbo),
      s1Wl, r1(s1bl), s1Wr, out_w, r1(out_b))


def _sc_body(dst_hbm, pd_hbm, out_hbm, idx_v, flags_v, pd_v, strip_v,
             outblk_v, shared, sem):
    t = lax.axis_index("s")

    cp = pltpu.make_async_copy(dst_hbm.at[pl.ds(t * _EPS, _EPS)], idx_v, sem)
    cp.start()
    pltpu.sync_copy(pd_hbm, pd_v)
    p0 = pd_v[0, :]
    dv = pd_v[1, :]

    zero16 = jnp.zeros((16,), jnp.float32)
    ones16 = jnp.ones((16,), jnp.float32)

    with jax.named_scope("ph_zero"):
        @plsc.parallel_loop(0, _NTP // 16, unroll=8)
        def _zero(i):
            flags_v[pl.ds(i * 16, 16)] = zero16
    with jax.named_scope("ph_idxwait"):
        cp.wait()

    with jax.named_scope("ph_scatter"):
        @plsc.parallel_loop(0, _EPS // 16, unroll=8)
        def _scatter(i):
            iv = idx_v[pl.ds(i * 16, 16)]
            plsc.store_scatter(flags_v, [iv], ones16)

    with jax.named_scope("ph_publish"):
        pltpu.sync_copy(flags_v, shared.at[t])
        plsc.subcore_barrier()

    base = t * _STRIDE
    with jax.named_scope("ph_gather"):
        pltpu.sync_copy(shared.at[:, pl.ds(base, _STRIDE)], strip_v)

    with jax.named_scope("ph_assemble"):
        _assemble(strip_v, outblk_v, p0, dv)

    cid = lax.axis_index("c")
    tail = NT - (_NSUB - 1) * _STRIDE
    half = _STRIDE // 2
    off = cid * half
    with jax.named_scope("ph_outwrite"):
        @pl.when(t < _NSUB - 1)
        def _():
            pltpu.sync_copy(outblk_v.at[pl.ds(off, half), :],
                            out_hbm.at[pl.ds(base + off, half), :])
        @pl.when((t == _NSUB - 1) & (cid == 0))
        def _():
            pltpu.sync_copy(outblk_v.at[pl.ds(0, half), :],
                            out_hbm.at[pl.ds(base, half), :])
        @pl.when((t == _NSUB - 1) & (cid == 1))
        def _():
            pltpu.sync_copy(outblk_v.at[pl.ds(half, tail - half), :],
                            out_hbm.at[pl.ds(base + half, tail - half), :])


def _assemble(strip_v, outblk_v, p0, dv):
    @plsc.parallel_loop(0, _STRIDE // 16, unroll=2)
    def _chunk(c):
        rows = [strip_v[r, pl.ds(c * 16, 16)] for r in range(_NSUB)]
        while len(rows) > 1:
            rows = [a + b for a, b in zip(rows[::2], rows[1::2])]
        ind = jnp.minimum(rows[0], 1.0)
        for n in range(16):
            outblk_v[c * 16 + n, :] = p0 + ind[n] * dv


def _sc_scatter_build():
    mesh = plsc.VectorSubcoreMesh(core_axis_name="c", subcore_axis_name="s")
    return pl.kernel(
        _sc_body,
        mesh=mesh,
        compiler_params=pltpu.CompilerParams(needs_layout_passes=False),
        out_type=jax.ShapeDtypeStruct((NT, OUT), jnp.float32),
        scratch_types=[
            pltpu.VMEM((_EPS,), jnp.int32),
            pltpu.VMEM((_NTP,), jnp.float32),
            pltpu.VMEM((2, OUT), jnp.float32),
            pltpu.VMEM((_NSUB, _STRIDE), jnp.float32),
            pltpu.VMEM((_STRIDE, OUT), jnp.float32),
            pltpu.VMEM_SHARED((_NSUB, _NTP), jnp.float32),
            pltpu.SemaphoreType.DMA,
        ],
    )


def kernel(x_a_cat, x_a_num, x_t, edge_a2t, edge_t2a, emb_cat, num_w, num_b,
           aWq, aWk, aWv, aWo, abq, abk, abv, abo,
           tWq, tWk, tWv, tWo, tbq, tbk, tbv, tbo,
           s1Wl, s1bl, s1Wr, s2Wl, s2bl, s2Wr, out_w, out_b):
    pd, dst = _tc_pd(edge_a2t.astype(jnp.int32), aWv, abv, aWo, abo,
                     tWv, tbv, tWo, tbo, s1Wl, s1bl, s1Wr, out_w, out_b)
    return _sc_scatter_build()(dst, pd)
